# baseline (speedup 1.0000x reference)
import functools

import jax
import jax.numpy as jnp
from jax import lax
from jax.experimental import pallas as pl
from jax.experimental.pallas import tpu as pltpu
from jax.experimental.pallas import tpu_sc as plsc

F = 26
V = 100000
E = 16
B = 16384
BC = 512
GRID = (V + BC - 1) // BC
SLEN = GRID * BC

_NC, _NS = 2, 16
_NW = _NC * _NS
_BPW = B // _NW


def _collapse_body(w1ref, w2ref, w3ref, bref, b1ref, b2ref, b3ref,
                   wpat_ref, wlin_ref, c0_ref):
    we = jnp.dot(w2ref[...], w3ref[...], preferred_element_type=jnp.float32)
    weff = jnp.dot(w1ref[...], we, preferred_element_type=jnp.float32)
    c0_ref[...] = (bref[...] * weff[442:443, :]
                   + jnp.dot(b1ref[...], we,
                             preferred_element_type=jnp.float32)
                   + jnp.dot(b2ref[...], w3ref[...],
                             preferred_element_type=jnp.float32)
                   + b3ref[...])
    wlin_ref[...] = weff[0:F, :]
    eye = (lax.broadcasted_iota(jnp.int32, (E, E), 0)
           == lax.broadcasted_iota(jnp.int32, (E, E), 1)).astype(jnp.float32)
    rows = [lax.dot_general(weff[F + E * f:F + E * (f + 1), :], eye,
                            (((0,), (0,)), ((), ())),
                            preferred_element_type=jnp.float32)
            for f in range(F)]
    wpat_ref[...] = jnp.concatenate(rows, axis=0)


def _collapse(W1, W2, W3, b2d, b1r, b2r, b3r):
    return pl.pallas_call(
        _collapse_body,
        out_shape=(jax.ShapeDtypeStruct((F, E), jnp.float32),
                   jax.ShapeDtypeStruct((F, 1), jnp.float32),
                   jax.ShapeDtypeStruct((1, 1), jnp.float32)),
    )(W1, W2, W3, b2d, b1r, b2r, b3r)


_CH = 128


def _table_body(vref, wref, wpat_ref, wlin_ref, c0_ref, oref):
    wpat = wpat_ref[...]
    wlin = wlin_ref[...]
    c0 = c0_ref[...]
    ones16 = jnp.ones((E, 1), jnp.float32)
    ones1 = jnp.ones((1, 1), jnp.float32)
    for c in range(BC // _CH):
        r = pl.ds(_CH * c, _CH)
        acc16 = vref[0, r, :] * wpat[0:1, :]
        accw = wref[0, r, :] * wlin[0:1, :]
        for f in range(1, F):
            acc16 = acc16 + vref[f, r, :] * wpat[f:f + 1, :]
            accw = accw + wref[f, r, :] * wlin[f:f + 1, :]
        t = (lax.dot_general(ones16, acc16, (((0,), (1,)), ((), ())),
                             preferred_element_type=jnp.float32)
             + lax.dot_general(ones1, accw, (((0,), (1,)), ((), ())),
                               preferred_element_type=jnp.float32))
        oref[pl.ds(_CH * c, _CH)] = jax.nn.sigmoid(t + c0).reshape(_CH)


def _build_table(v_tables, w_tables, wpat, wlin, c0):
    return pl.pallas_call(
        _table_body,
        grid=(GRID,),
        in_specs=[
            pl.BlockSpec((F, BC, E), lambda i: (0, i, 0)),
            pl.BlockSpec((F, BC, 1), lambda i: (0, i, 0)),
            pl.BlockSpec((F, E), lambda i: (0, 0)),
            pl.BlockSpec((F, 1), lambda i: (0, 0)),
            pl.BlockSpec((1, 1), lambda i: (0, 0)),
        ],
        out_specs=pl.BlockSpec((BC,), lambda i: (i,)),
        out_shape=jax.ShapeDtypeStruct((SLEN,), jnp.float32),
    )(v_tables, w_tables, wpat, wlin, c0)


def _gather_body(s_hbm, idx_hbm, out_hbm, idx_v, rows_v, sem):
    wid = lax.axis_index("s") * _NC + lax.axis_index("c")
    base = wid * _BPW
    pltpu.sync_copy(idx_hbm.at[pl.ds(base, _BPW)], idx_v)
    pltpu.async_copy(s_hbm.at[idx_v], rows_v, sem).wait()
    pltpu.sync_copy(rows_v, out_hbm.at[pl.ds(base, _BPW)])


def _gather(s2, idx):
    mesh = plsc.VectorSubcoreMesh(core_axis_name="c", subcore_axis_name="s")
    run = functools.partial(
        pl.kernel,
        mesh=mesh,
        out_type=jax.ShapeDtypeStruct((B,), jnp.float32),
        scratch_types=[
            pltpu.VMEM((_BPW,), jnp.int32),
            pltpu.VMEM((_BPW,), jnp.float32),
            pltpu.SemaphoreType.DMA,
        ],
    )(_gather_body)
    return run(s2, idx)


def kernel(inputs, w_tables, v_tables, b, W1, b1, W2, b2, W3, b3):
    wpat, wlin, c0 = _collapse(W1, W2, W3, b.reshape(1, 1),
                               b1.reshape(1, 256), b2.reshape(1, 128),
                               b3.reshape(1, 1))
    s = _build_table(v_tables, w_tables, wpat, wlin, c0)
    return _gather(s, inputs).reshape(B, 1)


# speedup vs baseline: 22.6591x; 22.6591x over previous
import functools

import jax
import jax.numpy as jnp
from jax import lax
from jax.experimental import pallas as pl
from jax.experimental.pallas import tpu as pltpu
from jax.experimental.pallas import tpu_sc as plsc

F = 26
V = 100000
E = 16
B = 16384
BCL = 2048
GRID = (V + BCL - 1) // BCL
SLEN = GRID * BCL

_NC, _NS = 2, 16
_NW = _NC * _NS
_BPW = B // _NW


def _collapse_body(w1ref, w2ref, w3ref, bref, b1ref, b2ref, b3ref,
                   wpat_ref, wlin_ref, c0_ref):
    we = jnp.dot(w2ref[...], w3ref[...], preferred_element_type=jnp.float32)
    weff = jnp.dot(w1ref[...], we, preferred_element_type=jnp.float32)
    c0_ref[...] = (bref[...] * weff[442:443, :]
                   + jnp.dot(b1ref[...], we,
                             preferred_element_type=jnp.float32)
                   + jnp.dot(b2ref[...], w3ref[...],
                             preferred_element_type=jnp.float32)
                   + b3ref[...])
    wlin_ref[...] = weff[0:F, :]
    wpat_ref[...] = jnp.concatenate(
        [weff[F + E * f:F + E * (f + 1), :] for f in range(F)], axis=1)


def _collapse(W1, W2, W3, b2d, b1r, b2r, b3r):
    return pl.pallas_call(
        _collapse_body,
        out_shape=(jax.ShapeDtypeStruct((E, F), jnp.float32),
                   jax.ShapeDtypeStruct((F, 1), jnp.float32),
                   jax.ShapeDtypeStruct((1, 1), jnp.float32)),
    )(W1, W2, W3, b2d, b1r, b2r, b3r)


def _table_body(vref, wref, wpat_ref, wlin_ref, c0_ref, oref):
    wpat = wpat_ref[...]
    wlin = wlin_ref[...]
    acc = vref[0] * wpat[:, 0:1]
    accw = wref[0] * wlin[0:1, 0:1]
    for f in range(1, F):
        acc = acc + vref[f] * wpat[:, f:f + 1]
        accw = accw + wref[f] * wlin[f:f + 1, 0:1]
    t = jnp.sum(acc, axis=0, keepdims=True) + accw + c0_ref[...]
    oref[...] = jax.nn.sigmoid(t).reshape(BCL)


def _build_table(vT, wT, wpat, wlin, c0):
    return pl.pallas_call(
        _table_body,
        grid=(GRID,),
        in_specs=[
            pl.BlockSpec((F, E, BCL), lambda i: (0, 0, i)),
            pl.BlockSpec((F, 1, BCL), lambda i: (0, 0, i)),
            pl.BlockSpec((E, F), lambda i: (0, 0)),
            pl.BlockSpec((F, 1), lambda i: (0, 0)),
            pl.BlockSpec((1, 1), lambda i: (0, 0)),
        ],
        out_specs=pl.BlockSpec((BCL,), lambda i: (i,)),
        out_shape=jax.ShapeDtypeStruct((SLEN,), jnp.float32),
    )(vT, wT, wpat, wlin, c0)


def _gather_body(s_hbm, idx_hbm, out_hbm, idx_v, rows_v, sem):
    wid = lax.axis_index("s") * _NC + lax.axis_index("c")
    base = wid * _BPW
    pltpu.sync_copy(idx_hbm.at[pl.ds(base, _BPW)], idx_v)
    pltpu.async_copy(s_hbm.at[idx_v], rows_v, sem).wait()
    pltpu.sync_copy(rows_v, out_hbm.at[pl.ds(base, _BPW)])


def _gather(s2, idx):
    mesh = plsc.VectorSubcoreMesh(core_axis_name="c", subcore_axis_name="s")
    run = functools.partial(
        pl.kernel,
        mesh=mesh,
        out_type=jax.ShapeDtypeStruct((B,), jnp.float32),
        scratch_types=[
            pltpu.VMEM((_BPW,), jnp.int32),
            pltpu.VMEM((_BPW,), jnp.float32),
            pltpu.SemaphoreType.DMA,
        ],
    )(_gather_body)
    return run(s2, idx)


def kernel(inputs, w_tables, v_tables, b, W1, b1, W2, b2, W3, b3):
    wpat, wlin, c0 = _collapse(W1, W2, W3, b.reshape(1, 1),
                               b1.reshape(1, 256), b2.reshape(1, 128),
                               b3.reshape(1, 1))
    vT = jnp.transpose(v_tables, (0, 2, 1))
    wT = jnp.transpose(w_tables, (0, 2, 1))
    s = _build_table(vT, wT, wpat, wlin, c0)
    return _gather(s, inputs).reshape(B, 1)


# speedup vs baseline: 25.6080x; 1.1301x over previous
import functools

import jax
import jax.numpy as jnp
from jax import lax
from jax.experimental import pallas as pl
from jax.experimental.pallas import tpu as pltpu
from jax.experimental.pallas import tpu_sc as plsc

F = 26
V = 100000
E = 16
B = 16384
BCL = 4096
GRID = (V + BCL - 1) // BCL
SLEN = GRID * BCL

_NC, _NS = 2, 16
_NW = _NC * _NS
_BPW = B // _NW


def _collapse_body(w1ref, w2ref, w3ref, bref, b1ref, b2ref, b3ref,
                   wpat_ref, wlin_ref, c0_ref):
    we = jnp.dot(w2ref[...], w3ref[...], preferred_element_type=jnp.float32)
    weff = jnp.dot(w1ref[...], we, preferred_element_type=jnp.float32)
    c0_ref[...] = (bref[...] * weff[442:443, :]
                   + jnp.dot(b1ref[...], we,
                             preferred_element_type=jnp.float32)
                   + jnp.dot(b2ref[...], w3ref[...],
                             preferred_element_type=jnp.float32)
                   + b3ref[...])
    wlin_ref[...] = weff[0:F, :]
    wpat_ref[...] = jnp.concatenate(
        [weff[F + E * f:F + E * (f + 1), :] for f in range(F)], axis=1)


def _collapse(W1, W2, W3, b2d, b1r, b2r, b3r):
    return pl.pallas_call(
        _collapse_body,
        out_shape=(jax.ShapeDtypeStruct((E, F), jnp.float32),
                   jax.ShapeDtypeStruct((F, 1), jnp.float32),
                   jax.ShapeDtypeStruct((1, 1), jnp.float32)),
    )(W1, W2, W3, b2d, b1r, b2r, b3r)


def _table_body(vref, wref, wpat_ref, wlin_ref, c0_ref, oref):
    wpat = wpat_ref[...]
    wlin = wlin_ref[...]
    acc = vref[0] * wpat[:, 0:1]
    accw = wref[0] * wlin[0:1, 0:1]
    for f in range(1, F):
        acc = acc + vref[f] * wpat[:, f:f + 1]
        accw = accw + wref[f] * wlin[f:f + 1, 0:1]
    t = jnp.sum(acc, axis=0, keepdims=True) + accw + c0_ref[...]
    oref[...] = jax.nn.sigmoid(t).reshape(BCL)


def _build_table(vT, wT, wpat, wlin, c0):
    return pl.pallas_call(
        _table_body,
        grid=(GRID,),
        in_specs=[
            pl.BlockSpec((F, E, BCL), lambda i: (0, 0, i)),
            pl.BlockSpec((F, 1, BCL), lambda i: (0, 0, i)),
            pl.BlockSpec((E, F), lambda i: (0, 0)),
            pl.BlockSpec((F, 1), lambda i: (0, 0)),
            pl.BlockSpec((1, 1), lambda i: (0, 0)),
        ],
        out_specs=pl.BlockSpec((BCL,), lambda i: (i,)),
        out_shape=jax.ShapeDtypeStruct((SLEN,), jnp.float32),
    )(vT, wT, wpat, wlin, c0)


def _gather_body(s_hbm, idx_hbm, out_hbm, idx_v, rows_v, sem):
    wid = lax.axis_index("s") * _NC + lax.axis_index("c")
    base = wid * _BPW
    pltpu.sync_copy(idx_hbm.at[pl.ds(base, _BPW)], idx_v)
    pltpu.async_copy(s_hbm.at[idx_v], rows_v, sem).wait()
    pltpu.sync_copy(rows_v, out_hbm.at[pl.ds(base, _BPW)])


def _gather(s2, idx):
    mesh = plsc.VectorSubcoreMesh(core_axis_name="c", subcore_axis_name="s")
    run = functools.partial(
        pl.kernel,
        mesh=mesh,
        out_type=jax.ShapeDtypeStruct((B,), jnp.float32),
        scratch_types=[
            pltpu.VMEM((_BPW,), jnp.int32),
            pltpu.VMEM((_BPW,), jnp.float32),
            pltpu.SemaphoreType.DMA,
        ],
    )(_gather_body)
    return run(s2, idx)


def kernel(inputs, w_tables, v_tables, b, W1, b1, W2, b2, W3, b3):
    wpat, wlin, c0 = _collapse(W1, W2, W3, b.reshape(1, 1),
                               b1.reshape(1, 256), b2.reshape(1, 128),
                               b3.reshape(1, 1))
    vT = jnp.transpose(v_tables, (0, 2, 1))
    wT = jnp.transpose(w_tables, (0, 2, 1))
    s = _build_table(vT, wT, wpat, wlin, c0)
    return _gather(s, inputs).reshape(B, 1)
